# baseline (speedup 1.0000x reference)
import functools
import math

import jax
import jax.numpy as jnp
from jax.experimental import pallas as pl
from jax.experimental.pallas import tpu as pltpu


def _conv_kernel(x_ref, f2_ref, o_ref, *pt_refs, K, H, W):
    KK = K * K
    HW = H * W
    half = K // 2
    x = x_ref[0]
    pos = jax.lax.broadcasted_iota(jnp.int32, (1, HW), 1)
    r = pos // W
    c = pos - r * W
    for kh in range(K):
        for kw in range(K):
            t = kh * K + kw
            off = (kh - half) * W + (kw - half)
            rolled = x if off == 0 else pltpu.roll(x, shift=(-off) % HW, axis=1)
            conds = []
            if kh - half < 0:
                conds.append(r >= half - kh)
            if kh - half > 0:
                conds.append(r < H - (kh - half))
            if kw - half < 0:
                conds.append(c >= half - kw)
            if kw - half > 0:
                conds.append(c < W - (kw - half))
            if conds:
                v = conds[0]
                for extra in conds[1:]:
                    v = jnp.logical_and(v, extra)
                rolled = jnp.where(v, rolled, 0.0)
            for lt in range(len(pt_refs)):
                pt_refs[lt][t::KK, :] = rolled[:, lt * 128:(lt + 1) * 128]

    f2 = f2_ref[...]
    accs = [jnp.dot(f2, pt[...], preferred_element_type=jnp.float32)
            for pt in pt_refs]
    o_ref[0] = jnp.concatenate(accs, axis=1)


def kernel(x, uu, vv, mask):
    B, d_i, H, W = x.shape
    KK = uu.shape[0]
    K = math.isqrt(KK)
    d_o = vv.shape[1] // d_i
    HW = H * W

    f2 = (uu @ vv + mask).reshape(d_o, d_i * KK)

    xf = x.reshape(B, d_i, HW)
    n_tiles = HW // 128
    out = pl.pallas_call(
        functools.partial(_conv_kernel, K=K, H=H, W=W),
        out_shape=jax.ShapeDtypeStruct((B, d_o, HW), jnp.float32),
        grid=(B,),
        in_specs=[
            pl.BlockSpec((1, d_i, HW), lambda i: (i, 0, 0)),
            pl.BlockSpec((d_o, d_i * KK), lambda i: (0, 0)),
        ],
        out_specs=pl.BlockSpec((1, d_o, HW), lambda i: (i, 0, 0)),
        scratch_shapes=[pltpu.VMEM((d_i * KK, 128), jnp.float32)
                        for _ in range(n_tiles)],
        compiler_params=pltpu.CompilerParams(
            dimension_semantics=("parallel",)),
    )(xf, f2)
    return out.reshape(B, d_o, H, W).astype(x.dtype)


# speedup vs baseline: 1.0366x; 1.0366x over previous
import functools
import math

import jax
import jax.numpy as jnp
from jax.experimental import pallas as pl
from jax.experimental.pallas import tpu as pltpu


def _conv_kernel(x_ref, f2_ref, o_ref, *pt_refs, K, H, W):
    KK = K * K
    HW = H * W
    half = K // 2
    x = x_ref[0]
    pos = jax.lax.broadcasted_iota(jnp.int32, (1, HW), 1)
    r = pos // W
    c = pos - r * W
    for kh in range(K):
        for kw in range(K):
            t = kh * K + kw
            off = (kh - half) * W + (kw - half)
            rolled = x if off == 0 else pltpu.roll(x, shift=(-off) % HW, axis=1)
            conds = []
            if kh - half < 0:
                conds.append(r >= half - kh)
            if kh - half > 0:
                conds.append(r < H - (kh - half))
            if kw - half < 0:
                conds.append(c >= half - kw)
            if kw - half > 0:
                conds.append(c < W - (kw - half))
            if conds:
                v = conds[0]
                for extra in conds[1:]:
                    v = jnp.logical_and(v, extra)
                rolled = jnp.where(v, rolled, 0.0)
            for lt in range(len(pt_refs)):
                pt_refs[lt][t::KK, :] = rolled[:, lt * 128:(lt + 1) * 128]

    f2 = f2_ref[...]
    pts = jnp.concatenate([pt[...] for pt in pt_refs], axis=1)
    o_ref[0] = jnp.dot(f2, pts, preferred_element_type=jnp.float32)


def kernel(x, uu, vv, mask):
    B, d_i, H, W = x.shape
    KK = uu.shape[0]
    K = math.isqrt(KK)
    d_o = vv.shape[1] // d_i
    HW = H * W

    f2 = (uu @ vv + mask).reshape(d_o, d_i * KK)

    xf = x.reshape(B, d_i, HW)
    n_tiles = HW // 128
    out = pl.pallas_call(
        functools.partial(_conv_kernel, K=K, H=H, W=W),
        out_shape=jax.ShapeDtypeStruct((B, d_o, HW), jnp.float32),
        grid=(B,),
        in_specs=[
            pl.BlockSpec((1, d_i, HW), lambda i: (i, 0, 0)),
            pl.BlockSpec((d_o, d_i * KK), lambda i: (0, 0)),
        ],
        out_specs=pl.BlockSpec((1, d_o, HW), lambda i: (i, 0, 0)),
        scratch_shapes=[pltpu.VMEM((d_i * KK, 128), jnp.float32)
                        for _ in range(n_tiles)],
        compiler_params=pltpu.CompilerParams(
            dimension_semantics=("parallel",)),
    )(xf, f2)
    return out.reshape(B, d_o, H, W).astype(x.dtype)
